# baseline (speedup 1.0000x reference)
import functools

import jax
import jax.numpy as jnp
from jax import lax
from jax.experimental import pallas as pl
from jax.experimental.pallas import tpu as pltpu
from jax.experimental.pallas import tpu_sc as plsc

NC = 2
NS = 16
NW = NC * NS
CH = 128
NBUF = 4


@functools.partial(jax.jit, static_argnames=("n_chunks", "d"))
def _sc_gather(idx3, params, n_chunks, d):
    btot = NW * n_chunks * CH
    mesh = plsc.VectorSubcoreMesh(core_axis_name="c", subcore_axis_name="s")

    @functools.partial(
        pl.kernel,
        mesh=mesh,
        out_type=jax.ShapeDtypeStruct((btot, d), jnp.float32),
        scratch_types=[
            pltpu.VMEM((n_chunks, CH), jnp.int32),
            pltpu.VMEM((NBUF, CH, d), jnp.float32),
        ]
        + [pltpu.SemaphoreType.DMA] * (2 * NBUF),
        compiler_params=pltpu.CompilerParams(use_tc_tiling_on_sc=False),
    )
    def k(table_hbm, idx_hbm, out_hbm, idx_v, rows_v, *sems):
        in_sems = sems[:NBUF]
        out_sems = sems[NBUF:]
        wid = lax.axis_index("s") * NC + lax.axis_index("c")
        base = wid * (n_chunks * CH)

        pltpu.sync_copy(idx_hbm.at[wid], idx_v)

        def gather(j, b):
            return pltpu.make_async_copy(
                table_hbm.at[idx_v.at[j]], rows_v.at[b], in_sems[b]
            )

        def put(j, b):
            return pltpu.make_async_copy(
                rows_v.at[b], out_hbm.at[pl.ds(base + j * CH, CH)], out_sems[b]
            )

        for b in range(NBUF):
            gather(b, b).start()

        def body(g, carry):
            for b in range(NBUF):
                j = g * NBUF + b
                gather(j, b).wait()
                put(j, b).start()
                put(j, b).wait()
                gather(j + NBUF, b).start()
            return carry

        lax.fori_loop(0, (n_chunks - NBUF) // NBUF, body, 0)

        for b in range(NBUF):
            j = (n_chunks - NBUF) + b
            gather(j, b).wait()
            put(j, b).start()
        for b in range(NBUF):
            j = (n_chunks - NBUF) + b
            put(j, b).wait()

    return k(params, idx3)


def kernel(token_index, params):
    b, t = token_index.shape
    d = params.shape[1]
    flat = token_index.reshape(-1).astype(jnp.int32)
    n_chunks = flat.shape[0] // (NW * CH)
    idx3 = flat.reshape(NW, n_chunks, CH)
    out = _sc_gather(idx3, params, n_chunks, d)
    return out.reshape(b, t, d)


# speedup vs baseline: 1.3278x; 1.3278x over previous
import functools

import jax
import jax.numpy as jnp
from jax import lax
from jax.experimental import pallas as pl
from jax.experimental.pallas import tpu as pltpu
from jax.experimental.pallas import tpu_sc as plsc

NC = 2
NS = 16
NW = NC * NS
CH = 128
NBUF = 4


@functools.partial(jax.jit, static_argnames=("n_chunks", "d"))
def _sc_gather(idx3, params2, n_chunks, d):
    btot = NW * n_chunks * CH
    mesh = plsc.VectorSubcoreMesh(core_axis_name="c", subcore_axis_name="s")

    @functools.partial(
        pl.kernel,
        mesh=mesh,
        out_type=jax.ShapeDtypeStruct((btot, 2 * d), jnp.float32),
        scratch_types=[
            pltpu.VMEM((n_chunks, CH), jnp.int32),
            pltpu.VMEM((NBUF, CH, d), jnp.float32),
        ]
        + [pltpu.SemaphoreType.DMA] * (2 * NBUF),
        compiler_params=pltpu.CompilerParams(use_tc_tiling_on_sc=False),
    )
    def k(table_hbm, idx_hbm, out_hbm, idx_v, rows_v, *sems):
        in_sems = sems[:NBUF]
        out_sems = sems[NBUF:]
        wid = lax.axis_index("s") * NC + lax.axis_index("c")
        base = wid * (n_chunks * CH)

        pltpu.sync_copy(idx_hbm.at[wid], idx_v)

        def gather(j, b):
            return pltpu.make_async_copy(
                table_hbm.at[idx_v.at[j]], rows_v.at[b], in_sems[b]
            )

        def put(j, b):
            return pltpu.make_async_copy(
                rows_v.at[b],
                out_hbm.at[pl.ds(base + j * CH, CH), pl.ds(0, d)],
                out_sems[b],
            )

        for b in range(NBUF):
            gather(b, b).start()

        def body(g, carry):
            for b in range(NBUF):
                j = g * NBUF + b
                gather(j, b).wait()
                put(j, b).start()
                put(j, b).wait()
                gather(j + NBUF, b).start()
            return carry

        lax.fori_loop(0, (n_chunks - NBUF) // NBUF, body, 0)

        for b in range(NBUF):
            j = (n_chunks - NBUF) + b
            gather(j, b).wait()
            put(j, b).start()
        for b in range(NBUF):
            j = (n_chunks - NBUF) + b
            put(j, b).wait()

    return k(params2, idx3)


def kernel(token_index, params):
    b, t = token_index.shape
    v, d = params.shape
    ph = lax.optimization_barrier(params.reshape(v // 2, 2 * d))
    table = ph.reshape(v, d)
    flat = token_index.reshape(-1).astype(jnp.int32)
    n_chunks = flat.shape[0] // (NW * CH)
    idx3 = flat.reshape(NW, n_chunks, CH)
    out = _sc_gather(idx3, table, n_chunks, d)
    return out[:, :d].reshape(b, t, d)
